# baseline (speedup 1.0000x reference)
import functools

import jax
import jax.numpy as jnp
from jax import lax
from jax.experimental import pallas as pl
from jax.experimental.pallas import tpu as pltpu
from jax.experimental.pallas import tpu_sc as plsc

B = 16384
F = 128
NC = 2
NS = 16
NW = NC * NS
BPW = B // NW
CH = 128
NCHUNK = BPW // CH


def _sc_gather(user, item, embed_user, embed_item, u_bias, i_bias):
    mesh = plsc.VectorSubcoreMesh(core_axis_name="c", subcore_axis_name="s")

    @functools.partial(
        pl.kernel,
        mesh=mesh,
        out_type=(
            jax.ShapeDtypeStruct((B, F), jnp.float32),
            jax.ShapeDtypeStruct((B, F), jnp.float32),
            jax.ShapeDtypeStruct((B,), jnp.float32),
            jax.ShapeDtypeStruct((B,), jnp.float32),
        ),
        scratch_types=[
            pltpu.VMEM((CH,), jnp.int32),
            pltpu.VMEM((CH,), jnp.int32),
            pltpu.VMEM((CH, F), jnp.float32),
            pltpu.VMEM((CH, F), jnp.float32),
            pltpu.VMEM((CH,), jnp.float32),
            pltpu.VMEM((CH,), jnp.float32),
            pltpu.SemaphoreType.DMA,
        ],
    )
    def k(user_h, item_h, eu_h, ei_h, ub_h, ib_h,
          eug_h, eig_h, ubg_h, ibg_h,
          idx_u, idx_i, ru, ri, bu, bi, sem):
        wid = lax.axis_index("s") * NC + lax.axis_index("c")
        base = wid * BPW
        for c in range(NCHUNK):
            off = base + c * CH
            pltpu.sync_copy(user_h.at[pl.ds(off, CH)], idx_u)
            pltpu.sync_copy(item_h.at[pl.ds(off, CH)], idx_i)
            cu = pltpu.async_copy(eu_h.at[idx_u], ru, sem)
            ci = pltpu.async_copy(ei_h.at[idx_i], ri, sem)
            cbu = pltpu.async_copy(ub_h.at[idx_u], bu, sem)
            cbi = pltpu.async_copy(ib_h.at[idx_i], bi, sem)
            cu.wait()
            ci.wait()
            cbu.wait()
            cbi.wait()
            pltpu.sync_copy(ru, eug_h.at[pl.ds(off, CH)])
            pltpu.sync_copy(ri, eig_h.at[pl.ds(off, CH)])
            pltpu.sync_copy(bu, ubg_h.at[pl.ds(off, CH)])
            pltpu.sync_copy(bi, ibg_h.at[pl.ds(off, CH)])

    return k(user, item, embed_user, embed_item, u_bias, i_bias)


def _tc_mlp(eu_g, ei_g, ub_g, ib_g, bias2, W0, b0, W1, b1, W2, b2, pred_w):
    BT = 1024

    def body(eu_ref, ei_ref, ub_ref, ib_ref, bias_ref,
             W0r, b0r, W1r, b1r, W2r, b2r, pwr, out_ref):
        x = eu_ref[...] * ei_ref[...]
        for Wr, br in ((W0r, b0r), (W1r, b1r), (W2r, b2r)):
            x = lax.dot_general(x, Wr[...], (((1,), (1,)), ((), ())),
                                preferred_element_type=jnp.float32)
            x = jnp.maximum(x + br[...], 0.0)
        x = x + (ub_ref[...] + ib_ref[...] + bias_ref[...])
        out_ref[...] = lax.dot_general(x, pwr[...], (((1,), (1,)), ((), ())),
                                       preferred_element_type=jnp.float32)

    full = lambda shape: pl.BlockSpec(shape, lambda i: (0, 0))
    out = pl.pallas_call(
        body,
        grid=(B // BT,),
        in_specs=[
            pl.BlockSpec((BT, F), lambda i: (i, 0)),
            pl.BlockSpec((BT, F), lambda i: (i, 0)),
            pl.BlockSpec((BT, 1), lambda i: (i, 0)),
            pl.BlockSpec((BT, 1), lambda i: (i, 0)),
            full((1, 1)),
            full((F, F)), full((1, F)),
            full((F, F)), full((1, F)),
            full((F, F)), full((1, F)),
            full((1, F)),
        ],
        out_specs=pl.BlockSpec((BT, 1), lambda i: (i, 0)),
        out_shape=jax.ShapeDtypeStruct((B, 1), jnp.float32),
    )(eu_g, ei_g, ub_g, ib_g, bias2,
      W0, b0.reshape(1, F), W1, b1.reshape(1, F), W2, b2.reshape(1, F),
      pred_w)
    return out


def kernel(user, item, embed_user, embed_item, u_bias, i_bias, bias_,
           W0, b0, W1, b1, W2, b2, pred_w):
    user = user.astype(jnp.int32)
    item = item.astype(jnp.int32)
    eu_g, ei_g, ub_g, ib_g = _sc_gather(
        user, item, embed_user, embed_item,
        u_bias.reshape(-1), i_bias.reshape(-1))
    pred = _tc_mlp(eu_g.reshape(B, F), ei_g.reshape(B, F),
                   ub_g.reshape(B, 1), ib_g.reshape(B, 1), bias_.reshape(1, 1),
                   W0, b0, W1, b1, W2, b2, pred_w)
    return pred.reshape(-1)
